# baseline (speedup 1.0000x reference)
import functools

import jax
import jax.numpy as jnp
from jax import lax
from jax.experimental import pallas as pl
from jax.experimental.pallas import tpu as pltpu
from jax.experimental.pallas import tpu_sc as plsc

BANK_N = 1024
V_LEN = 16
HID = 4096
BS = 128
NSLOT = BS * V_LEN
FLAT_N = BANK_N * V_LEN

ROWS_PER_TILE = 8


def _rotl(x, d):
    return lax.shift_left(x, jnp.uint32(d)) | lax.shift_right_logical(
        x, jnp.uint32(32 - d))


def _threefry_xor(x1):
    ks = (jnp.uint32(0), jnp.uint32(1), jnp.uint32(0x1BD11BDB))
    rot = ((13, 15, 26, 6), (17, 29, 16, 24))
    v0 = jnp.zeros_like(x1)
    v1 = x1 + ks[1]
    for i in range(5):
        for r in rot[i % 2]:
            v0 = v0 + v1
            v1 = _rotl(v1, r)
            v1 = v0 ^ v1
        v0 = v0 + ks[(i + 1) % 3]
        v1 = v1 + ks[(i + 2) % 3] + jnp.uint32(i + 1)
    return v0 ^ v1


def _sample_body(vid_ref, out_ref):
    s = pl.program_id(0)
    ib = pl.program_id(1)
    base = (s * BS + ib * ROWS_PER_TILE) * FLAT_N
    row = lax.broadcasted_iota(jnp.uint32, (ROWS_PER_TILE, FLAT_N), 0)
    col = lax.broadcasted_iota(jnp.uint32, (ROWS_PER_TILE, FLAT_N), 1)
    x1 = jnp.uint32(base) + row * jnp.uint32(FLAT_N) + col
    bits = _threefry_xor(x1)
    u = lax.shift_right_logical(bits, jnp.uint32(9)).astype(jnp.int32) + 1
    coli = col.astype(jnp.int32)
    start = vid_ref[:, :, 0].reshape(ROWS_PER_TILE, 1) * V_LEN
    banned = (coli >= start) & (coli < start + V_LEN)
    u = jnp.where(banned, 0, u)
    m = jnp.max(u, axis=1, keepdims=True)
    cand = jnp.where(u == m, coli, FLAT_N)
    first = jnp.min(cand, axis=1)
    out_ref[0, :, 0] = first


def _sample_call(vid_idx):
    grid = (V_LEN, BS // ROWS_PER_TILE)
    return pl.pallas_call(
        _sample_body,
        grid=grid,
        in_specs=[pl.BlockSpec((1, ROWS_PER_TILE, 1), lambda s, ib: (0, ib, 0))],
        out_specs=pl.BlockSpec((1, ROWS_PER_TILE, 1), lambda s, ib: (s, ib, 0)),
        out_shape=jax.ShapeDtypeStruct((V_LEN, BS, 1), jnp.int32),
    )(vid_idx.reshape(1, BS, 1))


_SC_WORKERS = 32
_ROWS_PER_W = NSLOT // _SC_WORKERS
_GROUP = 8


def _gather_blend_body(bank_hbm, idx_hbm, vid_hbm, bg_hbm, out_hbm,
                       idx_v, bg_v, bank_v, vid_v, sem):
    wid = lax.axis_index("s") * 2 + lax.axis_index("c")
    base = wid * _ROWS_PER_W
    pltpu.sync_copy(idx_hbm.at[pl.ds(base, _ROWS_PER_W)], idx_v)
    pltpu.sync_copy(bg_hbm.at[pl.ds(base, _ROWS_PER_W)], bg_v)
    for g in range(_ROWS_PER_W // _GROUP):
        r0 = g * _GROUP
        gath = pltpu.async_copy(
            bank_hbm.at[idx_v.at[pl.ds(r0, _GROUP)]], bank_v, sem)
        pltpu.sync_copy(vid_hbm.at[pl.ds(base + r0, _GROUP)], vid_v)
        gath.wait()
        for r in range(_GROUP):
            mvec = bg_v[r0 + r, :]
            inv = 1.0 - mvec

            def body(c, _):
                sl = pl.ds(c * 16, 16)
                bank_v[r, sl] = vid_v[r, sl] * inv + bank_v[r, sl] * mvec
                return 0

            lax.fori_loop(0, HID // 16, body, 0)
        pltpu.sync_copy(bank_v, out_hbm.at[pl.ds(base + r0, _GROUP)])


@functools.lru_cache(maxsize=1)
def _gather_blend():
    return functools.partial(
        pl.kernel,
        mesh=plsc.VectorSubcoreMesh(core_axis_name="c", subcore_axis_name="s"),
        out_type=jax.ShapeDtypeStruct((NSLOT, HID), jnp.float32),
        scratch_types=[
            pltpu.VMEM((_ROWS_PER_W,), jnp.int32),
            pltpu.VMEM((_ROWS_PER_W, 16), jnp.float32),
            pltpu.VMEM((_GROUP, HID), jnp.float32),
            pltpu.VMEM((_GROUP, HID), jnp.float32),
            pltpu.SemaphoreType.DMA,
        ],
    )(_gather_blend_body)


def kernel(bg_mask, vid_feats, vid_idx, mem_bank):
    bs, v_len, hid = vid_feats.shape
    n_bank = mem_bank.shape[0]
    sample_si = _sample_call(vid_idx.astype(jnp.int32))
    sample_flat = sample_si.reshape(v_len, bs).T.reshape(bs * v_len)
    bank_flat = mem_bank.reshape(n_bank * v_len, hid)
    vid_flat = vid_feats.reshape(bs * v_len, hid)
    bg_wide = jnp.broadcast_to(bg_mask.reshape(bs * v_len, 1), (bs * v_len, 16))
    out = _gather_blend()(bank_flat, sample_flat, vid_flat, bg_wide)
    return out.reshape(bs, v_len, hid)
